# baseline (speedup 1.0000x reference)
import functools

import jax
import jax.numpy as jnp
from jax import lax
from jax.experimental import pallas as pl
from jax.experimental.pallas import tpu as pltpu
from jax.experimental.pallas import tpu_sc as plsc

N_AGENTS = 100000
N_EDGES = 6400000
N_TIMESTEPS = 10
DELTA_T = 1.0
TAU = 0.1

NC = 2
NS = 16
NW = NC * NS
L = 16

N_PAD = 100352
SLICE = N_PAD // NS
EPT = 204800
E_PAD = EPT * NW
C = 3200
NCH = EPT // C
NB = 2

_mesh = plsc.VectorSubcoreMesh(core_axis_name="c", subcore_axis_name="s")


@functools.partial(
    pl.kernel,
    out_type=jax.ShapeDtypeStruct((NC * N_PAD,), jnp.float32),
    mesh=_mesh,
    compiler_params=pltpu.CompilerParams(needs_layout_passes=False),
    scratch_types=[
        pltpu.VMEM((N_PAD // 32,), jnp.int32),
        [pltpu.VMEM((C,), jnp.int32) for _ in range(NB)],
        [pltpu.VMEM((C,), jnp.int32) for _ in range(2 * NB)],
        [pltpu.VMEM((C,), jnp.float32) for _ in range(NB)],
        [pltpu.SemaphoreType.DMA for _ in range(NB)],
        [pltpu.SemaphoreType.DMA for _ in range(2 * NB)],
        [pltpu.SemaphoreType.DMA for _ in range(NB)],
        pltpu.VMEM_SHARED((N_PAD,), jnp.float32),
    ],
)
def _segsum_sc(bits_hbm, src_hbm, dst_hbm, zeros_hbm, out_hbm,
               bits_v, src_v, dst_v, vals_sc, dsrc, ddst, ssem, accum_sh):
    cid = lax.axis_index("c")
    sid = lax.axis_index("s")
    wid = sid * NC + cid
    base_e = wid * EPT
    s0 = sid * SLICE

    def issue_dma(j, k, kd):
        b = base_e + j * C
        pltpu.async_copy(src_hbm.at[pl.ds(b, C)], src_v[k], dsrc[k])
        pltpu.async_copy(dst_hbm.at[pl.ds(b, C)], dst_v[kd], ddst[kd])

    def wait_dma(j, k, kd):
        b = base_e + j * C
        pltpu.make_async_copy(src_hbm.at[pl.ds(b, C)], src_v[k], dsrc[k]).wait()
        pltpu.make_async_copy(dst_hbm.at[pl.ds(b, C)], dst_v[kd], ddst[kd]).wait()

    def gather(k):
        def g(m, _):
            sidx = src_v[k][pl.ds(m * L, L)]
            w = plsc.load_gather(bits_v, [lax.shift_right_logical(sidx, 5)])
            bit = lax.shift_right_logical(w, sidx & 31) & 1
            vals_sc[k][pl.ds(m * L, L)] = bit.astype(jnp.float32)
            return 0

        lax.fori_loop(0, C // L, g, 0, unroll=8)

    def issue_scatter(k, kd):
        pltpu.async_copy(vals_sc[k], accum_sh.at[dst_v[kd]], ssem[k], add=True)

    def wait_scatter(k, kd):
        pltpu.make_async_copy(vals_sc[k], accum_sh.at[dst_v[kd]], ssem[k]).wait()

    def body(j, k, kd, first):
        wait_dma(j, k, kd)
        if not first:
            wait_scatter(k, (kd + 2) % 4)
        gather(k)
        issue_scatter(k, kd)
        issue_dma(j + 2, k, (kd + 2) % 4)

    pltpu.sync_copy(bits_hbm, bits_v)
    pltpu.sync_copy(zeros_hbm.at[pl.ds(s0, SLICE)], accum_sh.at[pl.ds(s0, SLICE)])
    plsc.subcore_barrier()

    issue_dma(0, 0, 0)
    issue_dma(1, 1, 1)
    body(0, 0, 0, True)
    body(1, 1, 1, True)
    body(2, 0, 2, False)
    body(3, 1, 3, False)

    def group(p, _):
        j = 4 * p
        for q in range(4):
            body(j + q, q % 2, q, False)
        return 0

    lax.fori_loop(1, NCH // 4, group, 0)

    wait_scatter(0, 2)
    wait_scatter(1, 3)
    wait_dma(NCH, 0, 0)
    wait_dma(NCH + 1, 1, 1)

    plsc.subcore_barrier()
    pltpu.sync_copy(
        accum_sh.at[pl.ds(s0, SLICE)],
        out_hbm.at[pl.ds(cid * N_PAD + s0, SLICE)],
    )


def _sample_bgs(key, probs, tau=TAU):
    logits = jnp.log(jnp.stack([probs, 1.0 - probs], axis=1))
    g = jax.random.gumbel(key, logits.shape, dtype=logits.dtype)
    y_soft = jax.nn.softmax((logits + g) / tau, axis=1)
    idx = jnp.argmax(y_soft, axis=1)
    y_hard = jax.nn.one_hot(idx, 2, dtype=y_soft.dtype)
    y = jax.lax.stop_gradient(y_hard - y_soft) + y_soft
    return y[:, 0]


def _pack_bits(x_f32):
    xi = x_f32.astype(jnp.int32).reshape(N_PAD // 32, 32)
    return jnp.sum(xi << jnp.arange(32, dtype=jnp.int32)[None, :], axis=1,
                   dtype=jnp.int32)


def kernel(params, edge_index):
    gamma, rho, frac0 = params[0], params[1], params[2]
    base = jax.random.key(42)
    src = edge_index[0].astype(jnp.int32)
    dst = edge_index[1].astype(jnp.int32)
    n_extra = E_PAD + 2 * C - N_EDGES
    src = jnp.concatenate([src, jnp.full((n_extra,), N_AGENTS, jnp.int32)])
    dst = jnp.concatenate(
        [dst, N_AGENTS + (jnp.arange(n_extra, dtype=jnp.int32) % 256)]
    )
    zeros_pad = jnp.zeros((N_PAD,), jnp.float32)

    probs0 = frac0 * jnp.ones(N_AGENTS, dtype=jnp.float32)
    new_infected = _sample_bgs(jax.random.fold_in(base, 0), probs0)
    infected = new_infected
    susceptible = 1.0 - new_infected
    recovered = jnp.zeros(N_AGENTS, dtype=jnp.float32)

    def observe(inf, sus, rec):
        states = jnp.argmax(jnp.stack([inf, sus, rec], axis=0), axis=0)
        return inf.sum() / N_AGENTS, sus.sum() / N_AGENTS, rec.sum() / N_AGENTS, states

    i0, s0, r0, st0 = observe(infected, susceptible, recovered)
    inf_days, sus_days, rec_days, states_days = [i0], [s0], [r0], [st0]

    ones_bits = jnp.full((N_PAD // 32,), -1, jnp.int32)
    deg_parts = _segsum_sc(ones_bits, src, dst, zeros_pad)
    n_nb = deg_parts[:N_AGENTS] + deg_parts[N_PAD : N_PAD + N_AGENTS]

    zpad = jnp.zeros((N_PAD - N_AGENTS,), jnp.float32)
    for t in range(N_TIMESTEPS):
        bits = _pack_bits(jnp.concatenate([infected, zpad]))
        parts = _segsum_sc(bits, src, dst, zeros_pad)
        deg_inf = parts[:N_AGENTS] + parts[N_PAD : N_PAD + N_AGENTS]
        n_inf_nb = (1.0 - infected) * deg_inf
        lambda_ = (
            (susceptible + rho * recovered)
            * jnp.where(n_nb > 0, n_inf_nb / jnp.maximum(n_nb, 1.0), 0.0)
            * DELTA_T
        )
        p_inf = jnp.clip(1.0 - jnp.exp(-lambda_), 1e-10, 1.0)
        new_ir = _sample_bgs(jax.random.fold_in(base, 2 * t + 1), p_inf)
        p_rec = jnp.clip(gamma * infected, 1e-10, 1.0)
        new_rec = _sample_bgs(jax.random.fold_in(base, 2 * t + 2), p_rec)
        infected = infected + new_ir - new_rec
        susceptible = susceptible - susceptible * new_ir
        recovered = recovered + new_rec - recovered * new_ir
        i_t, s_t, r_t, st_t = observe(infected, susceptible, recovered)
        inf_days.append(i_t)
        sus_days.append(s_t)
        rec_days.append(r_t)
        states_days.append(st_t)

    return (
        jnp.stack(sus_days),
        jnp.stack(inf_days),
        jnp.stack(rec_days),
        jnp.stack(states_days),
    )


# speedup vs baseline: 1.1808x; 1.1808x over previous
import functools

import jax
import jax.numpy as jnp
from jax import lax
from jax.experimental import pallas as pl
from jax.experimental.pallas import tpu as pltpu
from jax.experimental.pallas import tpu_sc as plsc

N_AGENTS = 100000
N_EDGES = 6400000
N_TIMESTEPS = 10
DELTA_T = 1.0
TAU = 0.1

NC = 2
NS = 16
NW = NC * NS
L = 16

N_PAD = 100352
SLICE = N_PAD // NS
EPT = 204800
E_PAD = EPT * NW
C = 2048
NCH = EPT // C
NB = 2

_mesh = plsc.VectorSubcoreMesh(core_axis_name="c", subcore_axis_name="s")


@functools.partial(
    pl.kernel,
    out_type=jax.ShapeDtypeStruct((NC * N_PAD,), jnp.float32),
    mesh=_mesh,
    compiler_params=pltpu.CompilerParams(needs_layout_passes=False),
    scratch_types=[
        pltpu.VMEM((N_PAD,), jnp.float32),
        [pltpu.VMEM((C,), jnp.int32) for _ in range(NB)],
        [pltpu.VMEM((C,), jnp.int32) for _ in range(2 * NB)],
        [pltpu.VMEM((C,), jnp.float32) for _ in range(NB)],
        [pltpu.SemaphoreType.DMA for _ in range(NB)],
        [pltpu.SemaphoreType.DMA for _ in range(2 * NB)],
        [pltpu.SemaphoreType.DMA for _ in range(NB)],
        pltpu.VMEM_SHARED((N_PAD,), jnp.float32),
    ],
)
def _segsum_sc(inf_hbm, src_hbm, dst_hbm, zeros_hbm, out_hbm,
               inf_v, src_v, dst_v, vals_sc, dsrc, ddst, ssem, accum_sh):
    cid = lax.axis_index("c")
    sid = lax.axis_index("s")
    wid = sid * NC + cid
    base_e = wid * EPT
    s0 = sid * SLICE

    def issue_dma(j, k, kd):
        b = base_e + j * C
        pltpu.async_copy(src_hbm.at[pl.ds(b, C)], src_v[k], dsrc[k])
        pltpu.async_copy(dst_hbm.at[pl.ds(b, C)], dst_v[kd], ddst[kd])

    def wait_dma(j, k, kd):
        b = base_e + j * C
        pltpu.make_async_copy(src_hbm.at[pl.ds(b, C)], src_v[k], dsrc[k]).wait()
        pltpu.make_async_copy(dst_hbm.at[pl.ds(b, C)], dst_v[kd], ddst[kd]).wait()

    def gather(k):
        def g(m, _):
            idx = src_v[k][pl.ds(m * L, L)]
            vals_sc[k][pl.ds(m * L, L)] = plsc.load_gather(inf_v, [idx])
            return 0

        lax.fori_loop(0, C // L, g, 0, unroll=16)

    def issue_scatter(k, kd):
        pltpu.async_copy(vals_sc[k], accum_sh.at[dst_v[kd]], ssem[k], add=True)

    def wait_scatter(k, kd):
        pltpu.make_async_copy(vals_sc[k], accum_sh.at[dst_v[kd]], ssem[k]).wait()

    def body(j, k, kd, first):
        wait_dma(j, k, kd)
        if not first:
            wait_scatter(k, (kd + 2) % 4)
        gather(k)
        issue_scatter(k, kd)
        issue_dma(j + 2, k, (kd + 2) % 4)

    pltpu.sync_copy(inf_hbm, inf_v)
    pltpu.sync_copy(zeros_hbm.at[pl.ds(s0, SLICE)], accum_sh.at[pl.ds(s0, SLICE)])
    plsc.subcore_barrier()

    issue_dma(0, 0, 0)
    issue_dma(1, 1, 1)
    body(0, 0, 0, True)
    body(1, 1, 1, True)
    body(2, 0, 2, False)
    body(3, 1, 3, False)

    def group(p, _):
        j = 4 * p
        for q in range(4):
            body(j + q, q % 2, q, False)
        return 0

    lax.fori_loop(1, NCH // 4, group, 0)

    wait_scatter(0, 2)
    wait_scatter(1, 3)
    wait_dma(NCH, 0, 0)
    wait_dma(NCH + 1, 1, 1)

    plsc.subcore_barrier()
    pltpu.sync_copy(
        accum_sh.at[pl.ds(s0, SLICE)],
        out_hbm.at[pl.ds(cid * N_PAD + s0, SLICE)],
    )


def _sample_bgs(key, probs, tau=TAU):
    logits = jnp.log(jnp.stack([probs, 1.0 - probs], axis=1))
    g = jax.random.gumbel(key, logits.shape, dtype=logits.dtype)
    y_soft = jax.nn.softmax((logits + g) / tau, axis=1)
    idx = jnp.argmax(y_soft, axis=1)
    y_hard = jax.nn.one_hot(idx, 2, dtype=y_soft.dtype)
    y = jax.lax.stop_gradient(y_hard - y_soft) + y_soft
    return y[:, 0]


def kernel(params, edge_index):
    gamma, rho, frac0 = params[0], params[1], params[2]
    base = jax.random.key(42)
    src = edge_index[0].astype(jnp.int32)
    dst = edge_index[1].astype(jnp.int32)
    n_extra = E_PAD + 2 * C - N_EDGES
    src = jnp.concatenate([src, jnp.full((n_extra,), N_AGENTS, jnp.int32)])
    dst = jnp.concatenate(
        [dst, N_AGENTS + (jnp.arange(n_extra, dtype=jnp.int32) % 256)]
    )
    zeros_pad = jnp.zeros((N_PAD,), jnp.float32)

    probs0 = frac0 * jnp.ones(N_AGENTS, dtype=jnp.float32)
    new_infected = _sample_bgs(jax.random.fold_in(base, 0), probs0)
    infected = new_infected
    susceptible = 1.0 - new_infected
    recovered = jnp.zeros(N_AGENTS, dtype=jnp.float32)

    def observe(inf, sus, rec):
        states = jnp.argmax(jnp.stack([inf, sus, rec], axis=0), axis=0)
        return inf.sum() / N_AGENTS, sus.sum() / N_AGENTS, rec.sum() / N_AGENTS, states

    i0, s0, r0, st0 = observe(infected, susceptible, recovered)
    inf_days, sus_days, rec_days, states_days = [i0], [s0], [r0], [st0]

    ones_pad = jnp.ones((N_PAD,), jnp.float32)
    deg_parts = _segsum_sc(ones_pad, src, dst, zeros_pad)
    n_nb = deg_parts[:N_AGENTS] + deg_parts[N_PAD : N_PAD + N_AGENTS]

    for t in range(N_TIMESTEPS):
        inf_pad = jnp.concatenate(
            [infected, jnp.zeros((N_PAD - N_AGENTS,), jnp.float32)]
        )
        parts = _segsum_sc(inf_pad, src, dst, zeros_pad)
        deg_inf = parts[:N_AGENTS] + parts[N_PAD : N_PAD + N_AGENTS]
        n_inf_nb = (1.0 - infected) * deg_inf
        lambda_ = (
            (susceptible + rho * recovered)
            * jnp.where(n_nb > 0, n_inf_nb / jnp.maximum(n_nb, 1.0), 0.0)
            * DELTA_T
        )
        p_inf = jnp.clip(1.0 - jnp.exp(-lambda_), 1e-10, 1.0)
        new_ir = _sample_bgs(jax.random.fold_in(base, 2 * t + 1), p_inf)
        p_rec = jnp.clip(gamma * infected, 1e-10, 1.0)
        new_rec = _sample_bgs(jax.random.fold_in(base, 2 * t + 2), p_rec)
        infected = infected + new_ir - new_rec
        susceptible = susceptible - susceptible * new_ir
        recovered = recovered + new_rec - recovered * new_ir
        i_t, s_t, r_t, st_t = observe(infected, susceptible, recovered)
        inf_days.append(i_t)
        sus_days.append(s_t)
        rec_days.append(r_t)
        states_days.append(st_t)

    return (
        jnp.stack(sus_days),
        jnp.stack(inf_days),
        jnp.stack(rec_days),
        jnp.stack(states_days),
    )


# speedup vs baseline: 1.1816x; 1.0007x over previous
import functools

import jax
import jax.numpy as jnp
from jax import lax
from jax.experimental import pallas as pl
from jax.experimental.pallas import tpu as pltpu
from jax.experimental.pallas import tpu_sc as plsc

N_AGENTS = 100000
N_EDGES = 6400000
N_TIMESTEPS = 10
DELTA_T = 1.0
TAU = 0.1

NC = 2
NS = 16
NW = NC * NS
L = 16

N_PAD = 100352
SLICE = N_PAD // NS
EPT = 204800
E_PAD = EPT * NW
C = 2560
NCH = EPT // C
NB = 2

_mesh = plsc.VectorSubcoreMesh(core_axis_name="c", subcore_axis_name="s")


@functools.partial(
    pl.kernel,
    out_type=jax.ShapeDtypeStruct((NC * N_PAD,), jnp.float32),
    mesh=_mesh,
    compiler_params=pltpu.CompilerParams(needs_layout_passes=False),
    scratch_types=[
        pltpu.VMEM((N_PAD,), jnp.float32),
        [pltpu.VMEM((C,), jnp.int32) for _ in range(NB)],
        [pltpu.VMEM((C,), jnp.int32) for _ in range(2 * NB)],
        [pltpu.VMEM((C,), jnp.float32) for _ in range(NB)],
        [pltpu.SemaphoreType.DMA for _ in range(NB)],
        [pltpu.SemaphoreType.DMA for _ in range(2 * NB)],
        [pltpu.SemaphoreType.DMA for _ in range(NB)],
        pltpu.VMEM_SHARED((N_PAD,), jnp.float32),
    ],
)
def _segsum_sc(inf_hbm, src_hbm, dst_hbm, zeros_hbm, out_hbm,
               inf_v, src_v, dst_v, vals_sc, dsrc, ddst, ssem, accum_sh):
    cid = lax.axis_index("c")
    sid = lax.axis_index("s")
    wid = sid * NC + cid
    base_e = wid * EPT
    s0 = sid * SLICE

    def issue_dma(j, k, kd):
        b = base_e + j * C
        pltpu.async_copy(src_hbm.at[pl.ds(b, C)], src_v[k], dsrc[k])
        pltpu.async_copy(dst_hbm.at[pl.ds(b, C)], dst_v[kd], ddst[kd])

    def wait_dma(j, k, kd):
        b = base_e + j * C
        pltpu.make_async_copy(src_hbm.at[pl.ds(b, C)], src_v[k], dsrc[k]).wait()
        pltpu.make_async_copy(dst_hbm.at[pl.ds(b, C)], dst_v[kd], ddst[kd]).wait()

    def gather(k):
        def g(m, _):
            idx = src_v[k][pl.ds(m * L, L)]
            vals_sc[k][pl.ds(m * L, L)] = plsc.load_gather(inf_v, [idx])
            return 0

        lax.fori_loop(0, C // L, g, 0, unroll=16)

    def issue_scatter(k, kd):
        pltpu.async_copy(vals_sc[k], accum_sh.at[dst_v[kd]], ssem[k], add=True)

    def wait_scatter(k, kd):
        pltpu.make_async_copy(vals_sc[k], accum_sh.at[dst_v[kd]], ssem[k]).wait()

    def body(j, k, kd, first):
        wait_dma(j, k, kd)
        if not first:
            wait_scatter(k, (kd + 2) % 4)
        gather(k)
        issue_scatter(k, kd)
        issue_dma(j + 2, k, (kd + 2) % 4)

    pltpu.sync_copy(inf_hbm, inf_v)
    pltpu.sync_copy(zeros_hbm.at[pl.ds(s0, SLICE)], accum_sh.at[pl.ds(s0, SLICE)])
    plsc.subcore_barrier()

    issue_dma(0, 0, 0)
    issue_dma(1, 1, 1)
    body(0, 0, 0, True)
    body(1, 1, 1, True)
    body(2, 0, 2, False)
    body(3, 1, 3, False)

    def group(p, _):
        j = 4 * p
        for q in range(4):
            body(j + q, q % 2, q, False)
        return 0

    lax.fori_loop(1, NCH // 4, group, 0)

    wait_scatter(0, 2)
    wait_scatter(1, 3)
    wait_dma(NCH, 0, 0)
    wait_dma(NCH + 1, 1, 1)

    plsc.subcore_barrier()
    pltpu.sync_copy(
        accum_sh.at[pl.ds(s0, SLICE)],
        out_hbm.at[pl.ds(cid * N_PAD + s0, SLICE)],
    )


def _sample_bgs(key, probs, tau=TAU):
    logits = jnp.log(jnp.stack([probs, 1.0 - probs], axis=1))
    g = jax.random.gumbel(key, logits.shape, dtype=logits.dtype)
    y_soft = jax.nn.softmax((logits + g) / tau, axis=1)
    idx = jnp.argmax(y_soft, axis=1)
    y_hard = jax.nn.one_hot(idx, 2, dtype=y_soft.dtype)
    y = jax.lax.stop_gradient(y_hard - y_soft) + y_soft
    return y[:, 0]


def kernel(params, edge_index):
    gamma, rho, frac0 = params[0], params[1], params[2]
    base = jax.random.key(42)
    src = edge_index[0].astype(jnp.int32)
    dst = edge_index[1].astype(jnp.int32)
    n_extra = E_PAD + 2 * C - N_EDGES
    src = jnp.concatenate([src, jnp.full((n_extra,), N_AGENTS, jnp.int32)])
    dst = jnp.concatenate(
        [dst, N_AGENTS + (jnp.arange(n_extra, dtype=jnp.int32) % 256)]
    )
    zeros_pad = jnp.zeros((N_PAD,), jnp.float32)

    probs0 = frac0 * jnp.ones(N_AGENTS, dtype=jnp.float32)
    new_infected = _sample_bgs(jax.random.fold_in(base, 0), probs0)
    infected = new_infected
    susceptible = 1.0 - new_infected
    recovered = jnp.zeros(N_AGENTS, dtype=jnp.float32)

    def observe(inf, sus, rec):
        states = jnp.argmax(jnp.stack([inf, sus, rec], axis=0), axis=0)
        return inf.sum() / N_AGENTS, sus.sum() / N_AGENTS, rec.sum() / N_AGENTS, states

    i0, s0, r0, st0 = observe(infected, susceptible, recovered)
    inf_days, sus_days, rec_days, states_days = [i0], [s0], [r0], [st0]

    ones_pad = jnp.ones((N_PAD,), jnp.float32)
    deg_parts = _segsum_sc(ones_pad, src, dst, zeros_pad)
    n_nb = deg_parts[:N_AGENTS] + deg_parts[N_PAD : N_PAD + N_AGENTS]

    for t in range(N_TIMESTEPS):
        inf_pad = jnp.concatenate(
            [infected, jnp.zeros((N_PAD - N_AGENTS,), jnp.float32)]
        )
        parts = _segsum_sc(inf_pad, src, dst, zeros_pad)
        deg_inf = parts[:N_AGENTS] + parts[N_PAD : N_PAD + N_AGENTS]
        n_inf_nb = (1.0 - infected) * deg_inf
        lambda_ = (
            (susceptible + rho * recovered)
            * jnp.where(n_nb > 0, n_inf_nb / jnp.maximum(n_nb, 1.0), 0.0)
            * DELTA_T
        )
        p_inf = jnp.clip(1.0 - jnp.exp(-lambda_), 1e-10, 1.0)
        new_ir = _sample_bgs(jax.random.fold_in(base, 2 * t + 1), p_inf)
        p_rec = jnp.clip(gamma * infected, 1e-10, 1.0)
        new_rec = _sample_bgs(jax.random.fold_in(base, 2 * t + 2), p_rec)
        infected = infected + new_ir - new_rec
        susceptible = susceptible - susceptible * new_ir
        recovered = recovered + new_rec - recovered * new_ir
        i_t, s_t, r_t, st_t = observe(infected, susceptible, recovered)
        inf_days.append(i_t)
        sus_days.append(s_t)
        rec_days.append(r_t)
        states_days.append(st_t)

    return (
        jnp.stack(sus_days),
        jnp.stack(inf_days),
        jnp.stack(rec_days),
        jnp.stack(states_days),
    )
